# initial kernel scaffold (unmeasured)
import jax
import jax.numpy as jnp
from jax import lax
from jax.experimental import pallas as pl
from jax.experimental.pallas import tpu as pltpu

NY = 4
T = 4096
D = 2048
C = T // NY
NSTEP = 2 * (NY - 1)


def _ar_body(partial_ref, out_ref, send_buf, pchunk, stage, recv_bufs,
             send_sems, recv_sems, local_sem, out_sem):
    my_x = lax.axis_index("x")
    my_y = lax.axis_index("y")
    my_z = lax.axis_index("z")
    right = (my_x, (my_y + 1) % NY, my_z)
    left = (my_x, (my_y - 1) % NY, my_z)

    def rows(i):
        return pl.ds(i * C, C)

    first = pltpu.make_async_copy(partial_ref.at[rows(my_y)], send_buf, local_sem)
    first.start()

    barrier_sem = pltpu.get_barrier_semaphore()
    for nbr in (left, right):
        pl.semaphore_signal(barrier_sem, inc=1, device_id=nbr,
                            device_id_type=pl.DeviceIdType.MESH)
    pl.semaphore_wait(barrier_sem, 2)
    first.wait()

    for s in range(NY - 1):
        rdma = pltpu.make_async_remote_copy(
            src_ref=send_buf,
            dst_ref=recv_bufs.at[s],
            send_sem=send_sems.at[s],
            recv_sem=recv_sems.at[s],
            device_id=right,
            device_id_type=pl.DeviceIdType.MESH,
        )
        rdma.start()
        lcopy = pltpu.make_async_copy(
            partial_ref.at[rows((my_y - s - 1) % NY)], pchunk, local_sem)
        lcopy.start()
        lcopy.wait()
        rdma.wait()
        send_buf[...] = recv_bufs[s] + pchunk[...]

    stage[...] = send_buf[...].astype(jnp.float32)
    ocopy = pltpu.make_async_copy(stage, out_ref.at[rows((my_y + 1) % NY)], out_sem)
    ocopy.start()
    ocopy.wait()

    for s in range(NY - 1):
        src = send_buf if s == 0 else recv_bufs.at[NY - 2 + s]
        rdma = pltpu.make_async_remote_copy(
            src_ref=src,
            dst_ref=recv_bufs.at[NY - 1 + s],
            send_sem=send_sems.at[NY - 1 + s],
            recv_sem=recv_sems.at[NY - 1 + s],
            device_id=right,
            device_id_type=pl.DeviceIdType.MESH,
        )
        rdma.start()
        rdma.wait()
        stage[...] = recv_bufs[NY - 1 + s].astype(jnp.float32)
        ocopy = pltpu.make_async_copy(
            stage, out_ref.at[rows((my_y - s) % NY)], out_sem)
        ocopy.start()
        ocopy.wait()


def _all_reduce_y(partial):
    return pl.pallas_call(
        _ar_body,
        out_shape=jax.ShapeDtypeStruct((T, D), jnp.float32),
        in_specs=[pl.BlockSpec(memory_space=pltpu.MemorySpace.ANY)],
        out_specs=pl.BlockSpec(memory_space=pltpu.MemorySpace.ANY),
        scratch_shapes=[
            pltpu.VMEM((C, D), jnp.bfloat16),
            pltpu.VMEM((C, D), jnp.bfloat16),
            pltpu.VMEM((C, D), jnp.float32),
            pltpu.VMEM((NSTEP, C, D), jnp.bfloat16),
            pltpu.SemaphoreType.DMA((NSTEP,)),
            pltpu.SemaphoreType.DMA((NSTEP,)),
            pltpu.SemaphoreType.DMA,
            pltpu.SemaphoreType.DMA,
        ],
        compiler_params=pltpu.CompilerParams(collective_id=0),
    )(partial)


def kernel(ids, E):
    v_per = E.shape[0]
    my_y = lax.axis_index("y")
    local = ids - my_y * v_per
    mask = (local >= 0) & (local < v_per)
    safe = jnp.where(mask, local, 0)
    partial = jnp.where(mask[:, None], E[safe].astype(jnp.bfloat16),
                        jnp.bfloat16(0))
    return _all_reduce_y(partial)


# baseline (device time: 516254 ns/iter reference)
import jax
import jax.numpy as jnp
from jax import lax
from jax.experimental import pallas as pl
from jax.experimental.pallas import tpu as pltpu

NY = 4
T = 4096
D = 2048
C = T // NY
NSTEP = 2 * (NY - 1)


def _ar_body(partial_ref, out_ref, send_buf, pchunk, stage, recv_bufs,
             send_sems, recv_sems, local_sem, out_sem):
    my_x = lax.axis_index("x")
    my_y = lax.axis_index("y")
    my_z = lax.axis_index("z")
    right = (my_x, (my_y + 1) % NY, my_z)
    left = (my_x, (my_y - 1) % NY, my_z)

    def rows(i):
        return pl.ds(i * C, C)

    first = pltpu.make_async_copy(partial_ref.at[rows(my_y)], send_buf, local_sem)
    first.start()

    barrier_sem = pltpu.get_barrier_semaphore()
    for nbr in (left, right):
        pl.semaphore_signal(barrier_sem, inc=1, device_id=nbr,
                            device_id_type=pl.DeviceIdType.MESH)
    pl.semaphore_wait(barrier_sem, 2)
    first.wait()

    for s in range(NY - 1):
        rdma = pltpu.make_async_remote_copy(
            src_ref=send_buf,
            dst_ref=recv_bufs.at[s],
            send_sem=send_sems.at[s],
            recv_sem=recv_sems.at[s],
            device_id=right,
            device_id_type=pl.DeviceIdType.MESH,
        )
        rdma.start()
        lcopy = pltpu.make_async_copy(
            partial_ref.at[rows((my_y - s - 1) % NY)], pchunk, local_sem)
        lcopy.start()
        lcopy.wait()
        rdma.wait()
        send_buf[...] = recv_bufs[s] + pchunk[...]

    stage[...] = send_buf[...].astype(jnp.float32)
    ocopy = pltpu.make_async_copy(stage, out_ref.at[rows((my_y + 1) % NY)], out_sem)
    ocopy.start()
    ocopy.wait()

    for s in range(NY - 1):
        src = send_buf if s == 0 else recv_bufs.at[NY - 2 + s]
        rdma = pltpu.make_async_remote_copy(
            src_ref=src,
            dst_ref=recv_bufs.at[NY - 1 + s],
            send_sem=send_sems.at[NY - 1 + s],
            recv_sem=recv_sems.at[NY - 1 + s],
            device_id=right,
            device_id_type=pl.DeviceIdType.MESH,
        )
        rdma.start()
        rdma.wait()
        stage[...] = recv_bufs[NY - 1 + s].astype(jnp.float32)
        ocopy = pltpu.make_async_copy(
            stage, out_ref.at[rows((my_y - s) % NY)], out_sem)
        ocopy.start()
        ocopy.wait()


def _all_reduce_y(partial):
    return pl.pallas_call(
        _ar_body,
        out_shape=jax.ShapeDtypeStruct((T, D), jnp.float32),
        in_specs=[pl.BlockSpec(memory_space=pl.ANY)],
        out_specs=pl.BlockSpec(memory_space=pl.ANY),
        scratch_shapes=[
            pltpu.VMEM((C, D), jnp.bfloat16),
            pltpu.VMEM((C, D), jnp.bfloat16),
            pltpu.VMEM((C, D), jnp.float32),
            pltpu.VMEM((NSTEP, C, D), jnp.bfloat16),
            pltpu.SemaphoreType.DMA((NSTEP,)),
            pltpu.SemaphoreType.DMA((NSTEP,)),
            pltpu.SemaphoreType.DMA,
            pltpu.SemaphoreType.DMA,
        ],
        compiler_params=pltpu.CompilerParams(
            collective_id=0, vmem_limit_bytes=60 * 1024 * 1024),
    )(partial)


def kernel(ids, E):
    v_per = E.shape[0]
    my_y = lax.axis_index("y")
    local = ids - my_y * v_per
    mask = (local >= 0) & (local < v_per)
    safe = jnp.where(mask, local, 0)
    partial = jnp.where(mask[:, None], E[safe].astype(jnp.bfloat16),
                        jnp.bfloat16(0))
    return _all_reduce_y(partial)
